# Initial kernel scaffold; baseline (speedup 1.0000x reference)
import jax
import jax.numpy as jnp
from jax.experimental import pallas as pl


def kernel(x, edge_index, W1, b1, Wmu, bmu, Wls, bls, noise):
    raise NotImplementedError("write your pallas kernel here")


# speedup vs baseline: 10.7343x; 10.7343x over previous
import functools

import jax
import jax.numpy as jnp
from jax import lax
from jax.experimental import pallas as pl
from jax.experimental.pallas import tpu as pltpu
from jax.experimental.pallas import tpu_sc as plsc

N = 10000
E = 320000
D = 128
H = 128

NC = 2
NS = 16
LANES = 16
NW = NC * NS
CW = 128
TPR = 80
EP = NW * TPR * CW
TROWS = NW * TPR
SUBROWS = 632
LASTROWS = N - (NS - 1) * SUBROWS
NPAD = (NS - 1) * SUBROWS + 5 * CW

R = 2000
GRID = N // R


def _sc_degree(dst2):
  mesh = plsc.VectorSubcoreMesh(core_axis_name="c", subcore_axis_name="s")

  @functools.partial(
      pl.kernel,
      out_type=jax.ShapeDtypeStruct((NC * N, LANES), jnp.float32),
      mesh=mesh,
      scratch_types=[
          pltpu.VMEM((TPR, CW), jnp.int32),
          pltpu.VMEM((CW, LANES), jnp.float32),
          pltpu.VMEM((CW, LANES), jnp.float32),
          pltpu.VMEM_SHARED((NPAD, LANES), jnp.float32),
          pltpu.SemaphoreType.DMA,
      ],
  )
  def deg_kernel(dst_hbm, out_hbm, idx_v, ones_v, zeros_v, acc_sh, sem):
    cid = lax.axis_index("c")
    sid = lax.axis_index("s")
    wid = sid * NC + cid

    @pl.loop(0, CW)
    def _(i):
      ones_v[i, :] = jnp.ones((LANES,), jnp.float32)
      zeros_v[i, :] = jnp.zeros((LANES,), jnp.float32)

    @pl.loop(0, 5)
    def _(k):
      pltpu.sync_copy(zeros_v, acc_sh.at[pl.ds(sid * SUBROWS + k * CW, CW)])
    plsc.subcore_barrier()

    pltpu.sync_copy(dst_hbm.at[pl.ds(wid * TPR, TPR)], idx_v)

    @pl.loop(0, TPR)
    def _(j):
      pltpu.sync_copy(ones_v, acc_sh.at[idx_v.at[j]], add=True)

    plsc.subcore_barrier()

    @pl.when(sid < NS - 1)
    def _():
      pltpu.sync_copy(acc_sh.at[pl.ds(sid * SUBROWS, SUBROWS)],
                      out_hbm.at[pl.ds(cid * N + sid * SUBROWS, SUBROWS)])

    @pl.when(sid == NS - 1)
    def _():
      pltpu.sync_copy(acc_sh.at[pl.ds((NS - 1) * SUBROWS, LASTROWS)],
                      out_hbm.at[pl.ds(cid * N + (NS - 1) * SUBROWS,
                                       LASTROWS)])

  return deg_kernel(dst2)


def _sc_scatter(y, src2, dst2):
  mesh = plsc.VectorSubcoreMesh(core_axis_name="c", subcore_axis_name="s")

  @functools.partial(
      pl.kernel,
      out_type=jax.ShapeDtypeStruct((NC * N, D), jnp.float32),
      mesh=mesh,
      scratch_types=[
          pltpu.VMEM((TPR, CW), jnp.int32),
          pltpu.VMEM((TPR, CW), jnp.int32),
          pltpu.VMEM((CW, D), jnp.float32),
          pltpu.VMEM_SHARED((NPAD, D), jnp.float32),
          pltpu.SemaphoreType.DMA,
      ],
  )
  def scat_kernel(y_hbm, src_hbm, dst_hbm, out_hbm,
                  src_v, dst_v, rows_v, acc_sh, sem):
    cid = lax.axis_index("c")
    sid = lax.axis_index("s")
    wid = sid * NC + cid

    @pl.loop(0, CW)
    def _(i):
      @pl.loop(0, D, step=LANES)
      def _(d):
        rows_v[i, pl.ds(d, LANES)] = jnp.zeros((LANES,), jnp.float32)

    @pl.loop(0, 5)
    def _(k):
      pltpu.sync_copy(rows_v, acc_sh.at[pl.ds(sid * SUBROWS + k * CW, CW)])
    plsc.subcore_barrier()

    pltpu.sync_copy(src_hbm.at[pl.ds(wid * TPR, TPR)], src_v)
    pltpu.sync_copy(dst_hbm.at[pl.ds(wid * TPR, TPR)], dst_v)

    @pl.loop(0, TPR)
    def _(j):
      pltpu.sync_copy(y_hbm.at[src_v.at[j]], rows_v)
      pltpu.sync_copy(rows_v, acc_sh.at[dst_v.at[j]], add=True)

    plsc.subcore_barrier()

    @pl.when(sid < NS - 1)
    def _():
      pltpu.sync_copy(acc_sh.at[pl.ds(sid * SUBROWS, SUBROWS)],
                      out_hbm.at[pl.ds(cid * N + sid * SUBROWS, SUBROWS)])

    @pl.when(sid == NS - 1)
    def _():
      pltpu.sync_copy(acc_sh.at[pl.ds((NS - 1) * SUBROWS, LASTROWS)],
                      out_hbm.at[pl.ds(cid * N + (NS - 1) * SUBROWS,
                                       LASTROWS)])

  return scat_kernel(y, src2, dst2)


def _dinv_block(d0, d1):
  deg = d0[:, :1] + d1[:, :1] + 1.0
  return lax.rsqrt(deg), deg


def _tc_prep(degp, x):
  def body(d0, d1, x_ref, y_ref):
    dinv, _ = _dinv_block(d0[...], d1[...])
    y_ref[...] = x_ref[...] * dinv

  return pl.pallas_call(
      body,
      grid=(GRID,),
      in_specs=[
          pl.BlockSpec((R, LANES), lambda i: (i, 0)),
          pl.BlockSpec((R, LANES), lambda i: (i + GRID, 0)),
          pl.BlockSpec((R, D), lambda i: (i, 0)),
      ],
      out_specs=pl.BlockSpec((R, D), lambda i: (i, 0)),
      out_shape=jax.ShapeDtypeStruct((N, D), jnp.float32),
  )(degp, degp, x)


def _tc_layer1(degp, s1, x, W1, b1):
  def body(d0, d1, p0, p1, x_ref, w_ref, b_ref, h_ref, y2_ref):
    dinv, deg = _dinv_block(d0[...], d1[...])
    g = (p0[...] + p1[...]) * dinv + x_ref[...] * (1.0 / deg)
    h = jnp.dot(g, w_ref[...], preferred_element_type=jnp.float32) + b_ref[...]
    h = jnp.maximum(h, 0.0)
    h_ref[...] = h
    y2_ref[...] = h * dinv

  return pl.pallas_call(
      body,
      grid=(GRID,),
      in_specs=[
          pl.BlockSpec((R, LANES), lambda i: (i, 0)),
          pl.BlockSpec((R, LANES), lambda i: (i + GRID, 0)),
          pl.BlockSpec((R, D), lambda i: (i, 0)),
          pl.BlockSpec((R, D), lambda i: (i + GRID, 0)),
          pl.BlockSpec((R, D), lambda i: (i, 0)),
          pl.BlockSpec((D, H), lambda i: (0, 0)),
          pl.BlockSpec((1, H), lambda i: (0, 0)),
      ],
      out_specs=[
          pl.BlockSpec((R, H), lambda i: (i, 0)),
          pl.BlockSpec((R, H), lambda i: (i, 0)),
      ],
      out_shape=[
          jax.ShapeDtypeStruct((N, H), jnp.float32),
          jax.ShapeDtypeStruct((N, H), jnp.float32),
      ],
  )(degp, degp, s1, s1, x, W1, b1)


def _tc_layer2(degp, s2, h, Wmu, bmu, Wls, bls, noise):
  def body(d0, d1, p0, p1, h_ref, wm, bm, wl, bl, nz,
           z_ref, mu_ref, ls_ref):
    dinv, deg = _dinv_block(d0[...], d1[...])
    g = (p0[...] + p1[...]) * dinv + h_ref[...] * (1.0 / deg)
    mu = jnp.dot(g, wm[...], preferred_element_type=jnp.float32) + bm[...]
    ls = jnp.dot(g, wl[...], preferred_element_type=jnp.float32) + bl[...]
    mu_ref[...] = mu
    ls_ref[...] = ls
    z_ref[...] = mu + jnp.exp(ls) * nz[...]

  return pl.pallas_call(
      body,
      grid=(GRID,),
      in_specs=[
          pl.BlockSpec((R, LANES), lambda i: (i, 0)),
          pl.BlockSpec((R, LANES), lambda i: (i + GRID, 0)),
          pl.BlockSpec((R, H), lambda i: (i, 0)),
          pl.BlockSpec((R, H), lambda i: (i + GRID, 0)),
          pl.BlockSpec((R, H), lambda i: (i, 0)),
          pl.BlockSpec((H, H), lambda i: (0, 0)),
          pl.BlockSpec((1, H), lambda i: (0, 0)),
          pl.BlockSpec((H, H), lambda i: (0, 0)),
          pl.BlockSpec((1, H), lambda i: (0, 0)),
          pl.BlockSpec((R, H), lambda i: (i, 0)),
      ],
      out_specs=[
          pl.BlockSpec((R, H), lambda i: (i, 0)),
          pl.BlockSpec((R, H), lambda i: (i, 0)),
          pl.BlockSpec((R, H), lambda i: (i, 0)),
      ],
      out_shape=[
          jax.ShapeDtypeStruct((N, H), jnp.float32),
          jax.ShapeDtypeStruct((N, H), jnp.float32),
          jax.ShapeDtypeStruct((N, H), jnp.float32),
      ],
  )(degp, degp, s2, s2, h, Wmu, bmu, Wls, bls, noise)


def kernel(x, edge_index, W1, b1, Wmu, bmu, Wls, bls, noise):
  src = edge_index[0].astype(jnp.int32)
  dst = edge_index[1].astype(jnp.int32)
  pad = EP - E
  src2 = jnp.concatenate([src, jnp.zeros((pad,), jnp.int32)]).reshape(TROWS, CW)
  dst2 = jnp.concatenate([dst, jnp.full((pad,), N, jnp.int32)]).reshape(TROWS, CW)

  degp = _sc_degree(dst2)
  y1 = _tc_prep(degp, x)
  s1 = _sc_scatter(y1, src2, dst2)
  h, y2 = _tc_layer1(degp, s1, x, W1.astype(jnp.float32), b1.reshape(1, H))
  s2 = _sc_scatter(y2, src2, dst2)
  z, mu, logstd = _tc_layer2(degp, s2, h, Wmu, bmu.reshape(1, H),
                             Wls, bls.reshape(1, H), noise)
  return (z, mu, logstd)


# speedup vs baseline: 11.3416x; 1.0566x over previous
import functools

import jax
import jax.numpy as jnp
from jax import lax
from jax.experimental import pallas as pl
from jax.experimental.pallas import tpu as pltpu
from jax.experimental.pallas import tpu_sc as plsc

N = 10000
E = 320000
D = 128
H = 128

NC = 2
NS = 16
LANES = 16
NW = NC * NS
CW = 128
TPR = 80
EP = NW * TPR * CW
TROWS = NW * TPR
SUBROWS = 632
LASTROWS = N - (NS - 1) * SUBROWS
NPAD = N + 8

R = 2000
GRID = N // R


def _zero_acc(sid, zeros_v, acc_sh):
  @pl.loop(0, 4)
  def _(k):
    pltpu.sync_copy(zeros_v, acc_sh.at[pl.ds(sid * SUBROWS + k * CW, CW)])

  @pl.when(sid < NS - 1)
  def _():
    pltpu.sync_copy(zeros_v.at[pl.ds(0, SUBROWS - 4 * CW)],
                    acc_sh.at[pl.ds(sid * SUBROWS + 4 * CW, SUBROWS - 4 * CW)])

  @pl.when(sid == NS - 1)
  def _():
    pltpu.sync_copy(zeros_v.at[pl.ds(0, NPAD - (NS - 1) * SUBROWS - 4 * CW)],
                    acc_sh.at[pl.ds((NS - 1) * SUBROWS + 4 * CW,
                                    NPAD - (NS - 1) * SUBROWS - 4 * CW)])


def _sc_degree(dst2):
  mesh = plsc.VectorSubcoreMesh(core_axis_name="c", subcore_axis_name="s")

  @functools.partial(
      pl.kernel,
      out_type=jax.ShapeDtypeStruct((NC * N, LANES), jnp.float32),
      mesh=mesh,
      scratch_types=[
          pltpu.VMEM((TPR, CW), jnp.int32),
          pltpu.VMEM((CW, LANES), jnp.float32),
          pltpu.VMEM((CW, LANES), jnp.float32),
          pltpu.VMEM_SHARED((NPAD, LANES), jnp.float32),
          pltpu.SemaphoreType.DMA,
      ],
  )
  def deg_kernel(dst_hbm, out_hbm, idx_v, ones_v, zeros_v, acc_sh, sem):
    cid = lax.axis_index("c")
    sid = lax.axis_index("s")
    wid = sid * NC + cid

    @pl.loop(0, CW)
    def _(i):
      ones_v[i, :] = jnp.ones((LANES,), jnp.float32)
      zeros_v[i, :] = jnp.zeros((LANES,), jnp.float32)

    _zero_acc(sid, zeros_v, acc_sh)
    plsc.subcore_barrier()

    pltpu.sync_copy(dst_hbm.at[pl.ds(wid * TPR, TPR)], idx_v)

    @pl.loop(0, TPR)
    def _(j):
      pltpu.sync_copy(ones_v, acc_sh.at[idx_v.at[j]], add=True)

    plsc.subcore_barrier()

    @pl.when(sid < NS - 1)
    def _():
      pltpu.sync_copy(acc_sh.at[pl.ds(sid * SUBROWS, SUBROWS)],
                      out_hbm.at[pl.ds(cid * N + sid * SUBROWS, SUBROWS)])

    @pl.when(sid == NS - 1)
    def _():
      pltpu.sync_copy(acc_sh.at[pl.ds((NS - 1) * SUBROWS, LASTROWS)],
                      out_hbm.at[pl.ds(cid * N + (NS - 1) * SUBROWS,
                                       LASTROWS)])

  return deg_kernel(dst2)


NBUF = 2
PH = 2
TPP = TPR // PH
NITER = TPP // NBUF


def _sc_scatter(y, src2, dst2):
  mesh = plsc.VectorSubcoreMesh(core_axis_name="c", subcore_axis_name="s")

  @functools.partial(
      pl.kernel,
      out_type=jax.ShapeDtypeStruct((NC * N, D), jnp.float32),
      mesh=mesh,
      scratch_types=[
          pltpu.VMEM((TPP, CW), jnp.int32),
          pltpu.VMEM((TPP, CW), jnp.int32),
      ] + [pltpu.VMEM((CW, D), jnp.float32)] * NBUF + [
          pltpu.VMEM_SHARED((NPAD, D), jnp.float32),
      ] + [pltpu.SemaphoreType.DMA] * (2 * NBUF),
  )
  def scat_kernel(y_hbm, src_hbm, dst_hbm, out_hbm,
                  src_v, dst_v, *rest):
    rows = rest[:NBUF]
    acc_sh = rest[NBUF]
    gsem = rest[NBUF + 1:NBUF + 1 + NBUF]
    ssem = rest[NBUF + 1 + NBUF:]
    cid = lax.axis_index("c")
    sid = lax.axis_index("s")
    wid = sid * NC + cid

    @pl.loop(0, CW)
    def _(i):
      @pl.loop(0, D, step=LANES)
      def _(d):
        rows[0][i, pl.ds(d, LANES)] = jnp.zeros((LANES,), jnp.float32)

    _zero_acc(sid, rows[0], acc_sh)
    plsc.subcore_barrier()

    for ph in range(PH):
      base = wid * TPR + ph * TPP
      pltpu.sync_copy(src_hbm.at[pl.ds(base, TPP)], src_v)
      pltpu.sync_copy(dst_hbm.at[pl.ds(base, TPP)], dst_v)

      for b in range(NBUF):
        pltpu.async_copy(y_hbm.at[src_v.at[b]], rows[b], gsem[b])

      @pl.loop(0, NITER)
      def _(it):
        g = it * NBUF
        for b in range(NBUF):
          j = g + b
          pltpu.make_async_copy(y_hbm.at[src_v.at[j]], rows[b],
                                gsem[b]).wait()
          pltpu.async_copy(rows[b], acc_sh.at[dst_v.at[j]], ssem[b],
                           add=True)

        @pl.when(it < NITER - 1)
        def _():
          for b in range(NBUF):
            j = g + b
            pltpu.make_async_copy(rows[b], acc_sh.at[dst_v.at[j]],
                                  ssem[b]).wait()
            pltpu.async_copy(y_hbm.at[src_v.at[g + NBUF + b]], rows[b],
                             gsem[b])

      for b in range(NBUF):
        j = (NITER - 1) * NBUF + b
        pltpu.make_async_copy(rows[b], acc_sh.at[dst_v.at[j]], ssem[b]).wait()

    plsc.subcore_barrier()

    @pl.when(sid < NS - 1)
    def _():
      pltpu.sync_copy(acc_sh.at[pl.ds(sid * SUBROWS, SUBROWS)],
                      out_hbm.at[pl.ds(cid * N + sid * SUBROWS, SUBROWS)])

    @pl.when(sid == NS - 1)
    def _():
      pltpu.sync_copy(acc_sh.at[pl.ds((NS - 1) * SUBROWS, LASTROWS)],
                      out_hbm.at[pl.ds(cid * N + (NS - 1) * SUBROWS,
                                       LASTROWS)])

  return scat_kernel(y, src2, dst2)


def _dinv_block(d0, d1):
  deg = d0[:, :1] + d1[:, :1] + 1.0
  return lax.rsqrt(deg), deg


def _tc_prep(degp, x):
  def body(d0, d1, x_ref, y_ref):
    dinv, _ = _dinv_block(d0[...], d1[...])
    y_ref[...] = x_ref[...] * dinv

  return pl.pallas_call(
      body,
      grid=(GRID,),
      in_specs=[
          pl.BlockSpec((R, LANES), lambda i: (i, 0)),
          pl.BlockSpec((R, LANES), lambda i: (i + GRID, 0)),
          pl.BlockSpec((R, D), lambda i: (i, 0)),
      ],
      out_specs=pl.BlockSpec((R, D), lambda i: (i, 0)),
      out_shape=jax.ShapeDtypeStruct((N, D), jnp.float32),
  )(degp, degp, x)


def _tc_layer1(degp, s1, x, W1, b1):
  def body(d0, d1, p0, p1, x_ref, w_ref, b_ref, h_ref, y2_ref):
    dinv, deg = _dinv_block(d0[...], d1[...])
    g = (p0[...] + p1[...]) * dinv + x_ref[...] * (1.0 / deg)
    h = jnp.dot(g, w_ref[...], preferred_element_type=jnp.float32) + b_ref[...]
    h = jnp.maximum(h, 0.0)
    h_ref[...] = h
    y2_ref[...] = h * dinv

  return pl.pallas_call(
      body,
      grid=(GRID,),
      in_specs=[
          pl.BlockSpec((R, LANES), lambda i: (i, 0)),
          pl.BlockSpec((R, LANES), lambda i: (i + GRID, 0)),
          pl.BlockSpec((R, D), lambda i: (i, 0)),
          pl.BlockSpec((R, D), lambda i: (i + GRID, 0)),
          pl.BlockSpec((R, D), lambda i: (i, 0)),
          pl.BlockSpec((D, H), lambda i: (0, 0)),
          pl.BlockSpec((1, H), lambda i: (0, 0)),
      ],
      out_specs=[
          pl.BlockSpec((R, H), lambda i: (i, 0)),
          pl.BlockSpec((R, H), lambda i: (i, 0)),
      ],
      out_shape=[
          jax.ShapeDtypeStruct((N, H), jnp.float32),
          jax.ShapeDtypeStruct((N, H), jnp.float32),
      ],
  )(degp, degp, s1, s1, x, W1, b1)


def _tc_layer2(degp, s2, h, Wmu, bmu, Wls, bls, noise):
  def body(d0, d1, p0, p1, h_ref, wm, bm, wl, bl, nz,
           z_ref, mu_ref, ls_ref):
    dinv, deg = _dinv_block(d0[...], d1[...])
    g = (p0[...] + p1[...]) * dinv + h_ref[...] * (1.0 / deg)
    mu = jnp.dot(g, wm[...], preferred_element_type=jnp.float32) + bm[...]
    ls = jnp.dot(g, wl[...], preferred_element_type=jnp.float32) + bl[...]
    mu_ref[...] = mu
    ls_ref[...] = ls
    z_ref[...] = mu + jnp.exp(ls) * nz[...]

  return pl.pallas_call(
      body,
      grid=(GRID,),
      in_specs=[
          pl.BlockSpec((R, LANES), lambda i: (i, 0)),
          pl.BlockSpec((R, LANES), lambda i: (i + GRID, 0)),
          pl.BlockSpec((R, H), lambda i: (i, 0)),
          pl.BlockSpec((R, H), lambda i: (i + GRID, 0)),
          pl.BlockSpec((R, H), lambda i: (i, 0)),
          pl.BlockSpec((H, H), lambda i: (0, 0)),
          pl.BlockSpec((1, H), lambda i: (0, 0)),
          pl.BlockSpec((H, H), lambda i: (0, 0)),
          pl.BlockSpec((1, H), lambda i: (0, 0)),
          pl.BlockSpec((R, H), lambda i: (i, 0)),
      ],
      out_specs=[
          pl.BlockSpec((R, H), lambda i: (i, 0)),
          pl.BlockSpec((R, H), lambda i: (i, 0)),
          pl.BlockSpec((R, H), lambda i: (i, 0)),
      ],
      out_shape=[
          jax.ShapeDtypeStruct((N, H), jnp.float32),
          jax.ShapeDtypeStruct((N, H), jnp.float32),
          jax.ShapeDtypeStruct((N, H), jnp.float32),
      ],
  )(degp, degp, s2, s2, h, Wmu, bmu, Wls, bls, noise)


def kernel(x, edge_index, W1, b1, Wmu, bmu, Wls, bls, noise):
  src = edge_index[0].astype(jnp.int32)
  dst = edge_index[1].astype(jnp.int32)
  pad = EP - E
  src2 = jnp.concatenate([src, jnp.zeros((pad,), jnp.int32)]).reshape(TROWS, CW)
  dst2 = jnp.concatenate([dst, jnp.full((pad,), N, jnp.int32)]).reshape(TROWS, CW)

  degp = _sc_degree(dst2)
  y1 = _tc_prep(degp, x)
  s1 = _sc_scatter(y1, src2, dst2)
  h, y2 = _tc_layer1(degp, s1, x, W1.astype(jnp.float32), b1.reshape(1, H))
  s2 = _sc_scatter(y2, src2, dst2)
  z, mu, logstd = _tc_layer2(degp, s2, h, Wmu, bmu.reshape(1, H),
                             Wls, bls.reshape(1, H), noise)
  return (z, mu, logstd)
